# baseline (speedup 1.0000x reference)
import functools

import jax
import jax.numpy as jnp
from jax import lax
from jax.experimental import pallas as pl
from jax.experimental.pallas import tpu as pltpu
from jax.experimental.pallas import tpu_sc as plsc

N_LAYERS = 4
N_NODES = 10000
N_EDGES = 320000
D = 128
DE = 16
N_GRAPHS = 200

NC = 2
NS = 16
NW = NC * NS

E_PAD = 327680
EPW = E_PAD // NW
ECHK = 64
NCHK = EPW // ECHK
ACC_ROWS = 10240
RPT = ACC_ROWS // NS

RNODES = 312
RBUF = 328
RCHK = 128
RTAIL = N_NODES - (NW - 1) * RNODES - 2 * RCHK


def _sc_edge_kernel(hw_hbm, ec_hbm, src_hbm, dst_hbm, out_hbm,
                    srcA, srcB, dstA, dstB, ecA, ecB, rowsA, rowsB, acc,
                    sIA, sIB, sGA, sGB, sEA, sEB):
    cid = lax.axis_index("c")
    sid = lax.axis_index("s")
    w = sid * NC + cid

    src_bufs = (srcA, srcB)
    dst_bufs = (dstA, dstB)
    ec_bufs = (ecA, ecB)
    rows_bufs = (rowsA, rowsB)
    sI = (sIA, sIB)
    sG = (sGA, sGB)
    sE = (sEA, sEB)

    def _z(i, _):
        for j in range(D // 16):
            rowsA[i, pl.ds(j * 16, 16)] = jnp.zeros((16,), jnp.float32)
        return 0
    lax.fori_loop(0, ECHK, _z, 0)
    for k in range(RPT // ECHK):
        pltpu.sync_copy(rowsA, acc.at[pl.ds(sid * RPT + k * ECHK, ECHK)])
    plsc.subcore_barrier()

    base = w * EPW

    def issue_idx(c, b):
        e0 = base + c * ECHK
        pltpu.async_copy(src_hbm.at[pl.ds(e0, ECHK)], src_bufs[b], sI[b])
        pltpu.async_copy(dst_hbm.at[pl.ds(e0, ECHK)], dst_bufs[b], sI[b])

    def wait_idx(b):
        pltpu.make_async_copy(src_hbm.at[pl.ds(0, ECHK)], src_bufs[b],
                              sI[b]).wait()
        pltpu.make_async_copy(dst_hbm.at[pl.ds(0, ECHK)], dst_bufs[b],
                              sI[b]).wait()

    def adjust_and_issue(c, b):
        pltpu.async_copy(hw_hbm.at[src_bufs[b]], rows_bufs[b], sG[b])
        e0 = base + c * ECHK
        pltpu.async_copy(ec_hbm.at[pl.ds(e0, ECHK)], ec_bufs[b], sE[b])

    def wait_ge(b):
        pltpu.make_async_copy(hw_hbm.at[src_bufs[b]], rows_bufs[b],
                              sG[b]).wait()
        pltpu.make_async_copy(ec_hbm.at[pl.ds(0, ECHK)], ec_bufs[b],
                              sE[b]).wait()

    def compute(b):
        def _row(i, _):
            for j in range(D // 16):
                sl = pl.ds(j * 16, 16)
                rows_bufs[b][i, sl] = jnp.maximum(
                    rows_bufs[b][i, sl] + ec_bufs[b][i, sl], 0.0)
            return 0
        lax.fori_loop(0, ECHK, _row, 0)

    def scatter(b):
        pltpu.sync_copy(rows_bufs[b], acc.at[dst_bufs[b]], add=True)

    issue_idx(0, 0)
    wait_idx(0)
    adjust_and_issue(0, 0)
    issue_idx(1, 1)

    def _steady(i, _):
        for b in range(2):
            c = 2 * i + b
            b2 = 1 - b
            wait_ge(b)
            compute(b)
            wait_idx(b2)
            adjust_and_issue(c + 1, b2)
            issue_idx(c + 2, b)
            scatter(b)
        return 0
    lax.fori_loop(0, (NCHK - 2) // 2, _steady, 0)

    wait_ge(0)
    compute(0)
    wait_idx(1)
    adjust_and_issue(NCHK - 1, 1)
    scatter(0)
    wait_ge(1)
    compute(1)
    scatter(1)

    plsc.subcore_barrier()
    pltpu.sync_copy(acc.at[pl.ds(sid * RPT, RPT)],
                    out_hbm.at[cid, pl.ds(sid * RPT, RPT)])


_sc_edge = functools.partial(
    pl.kernel,
    mesh=plsc.VectorSubcoreMesh(core_axis_name="c", subcore_axis_name="s"),
    out_type=jax.ShapeDtypeStruct((NC, ACC_ROWS, D), jnp.float32),
    scratch_types=[
        pltpu.VMEM((ECHK,), jnp.int32),
        pltpu.VMEM((ECHK,), jnp.int32),
        pltpu.VMEM((ECHK,), jnp.int32),
        pltpu.VMEM((ECHK,), jnp.int32),
        pltpu.VMEM((ECHK, D), jnp.float32),
        pltpu.VMEM((ECHK, D), jnp.float32),
        pltpu.VMEM((ECHK, D), jnp.float32),
        pltpu.VMEM((ECHK, D), jnp.float32),
        pltpu.VMEM_SHARED((ACC_ROWS, D), jnp.float32),
        pltpu.SemaphoreType.DMA,
        pltpu.SemaphoreType.DMA,
        pltpu.SemaphoreType.DMA,
        pltpu.SemaphoreType.DMA,
        pltpu.SemaphoreType.DMA,
        pltpu.SemaphoreType.DMA,
    ],
    compiler_params=pltpu.CompilerParams(needs_layout_passes=False),
)(_sc_edge_kernel)


def _sc_readout_kernel(h_hbm, hw_hbm, ids_hbm, psum_hbm, pmax_hbm,
                       hbuf, wbuf, idsbuf, pS, pM):
    cid = lax.axis_index("c")
    sid = lax.axis_index("s")
    w = sid * NC + cid
    base = w * RNODES
    cnt = jnp.where(w == NW - 1, RBUF, RNODES)

    pltpu.sync_copy(ids_hbm.at[pl.ds(base, RBUF)], idsbuf)

    def _z(i, _):
        for j in range(D // 16):
            sl = pl.ds(j * 16, 16)
            pS[i, sl] = jnp.zeros((16,), jnp.float32)
            pM[i, sl] = jnp.zeros((16,), jnp.float32)
        return 0
    lax.fori_loop(0, N_GRAPHS, _z, 0)

    for c in range(3):
        start = base + c * RCHK
        if c < 2:
            pltpu.sync_copy(h_hbm.at[pl.ds(start, RCHK)], hbuf)
            pltpu.sync_copy(hw_hbm.at[pl.ds(start, RCHK)], wbuf)
        else:
            @pl.when(w < NW - 1)
            def _full():
                pltpu.sync_copy(h_hbm.at[pl.ds(start, RCHK)], hbuf)
                pltpu.sync_copy(hw_hbm.at[pl.ds(start, RCHK)], wbuf)

            @pl.when(w == NW - 1)
            def _tail():
                pltpu.sync_copy(h_hbm.at[pl.ds(start, RTAIL)],
                                hbuf.at[pl.ds(0, RTAIL)])
                pltpu.sync_copy(hw_hbm.at[pl.ds(start, RTAIL)],
                                wbuf.at[pl.ds(0, RTAIL)])

        rem = jnp.clip(cnt - c * RCHK, 0, RCHK)

        def _node(n, _):
            g16 = plsc.load_gather(
                idsbuf, [jnp.full((16,), c * RCHK + n, jnp.int32)])
            for j in range(D // 16):
                sl = pl.ds(j * 16, 16)
                cols = lax.iota(jnp.int32, 16) + j * 16
                s_old = plsc.load_gather(pS, [g16, cols])
                plsc.store_scatter(pS, [g16, cols], s_old + wbuf[n, sl])
                m_old = plsc.load_gather(pM, [g16, cols])
                plsc.store_scatter(pM, [g16, cols],
                                   jnp.maximum(m_old, hbuf[n, sl]))
            return 0
        lax.fori_loop(0, rem, _node, 0)

    pltpu.sync_copy(pS, psum_hbm.at[w])
    pltpu.sync_copy(pM, pmax_hbm.at[w])


_sc_readout = functools.partial(
    pl.kernel,
    mesh=plsc.VectorSubcoreMesh(core_axis_name="c", subcore_axis_name="s"),
    out_type=(
        jax.ShapeDtypeStruct((NW, N_GRAPHS, D), jnp.float32),
        jax.ShapeDtypeStruct((NW, N_GRAPHS, D), jnp.float32),
    ),
    scratch_types=[
        pltpu.VMEM((RCHK, D), jnp.float32),
        pltpu.VMEM((RCHK, D), jnp.float32),
        pltpu.VMEM((RBUF,), jnp.int32),
        pltpu.VMEM((N_GRAPHS, D), jnp.float32),
        pltpu.VMEM((N_GRAPHS, D), jnp.float32),
    ],
    compiler_params=pltpu.CompilerParams(needs_layout_passes=False),
)(_sc_readout_kernel)


def _tc_in_body(x_ref, win_ref, bin_ref, wh_ref, h_ref, hw_ref):
    h = jnp.maximum(
        jnp.dot(x_ref[...], win_ref[...], preferred_element_type=jnp.float32)
        + bin_ref[...], 0.0)
    h_ref[...] = h
    hw_ref[...] = jnp.dot(h, wh_ref[...], preferred_element_type=jnp.float32)


def _tc_in(x, W_in, b_in, Wh):
    return pl.pallas_call(
        _tc_in_body,
        out_shape=(
            jax.ShapeDtypeStruct((N_NODES, D), jnp.float32),
            jax.ShapeDtypeStruct((N_NODES, D), jnp.float32),
        ),
    )(x, W_in, b_in, Wh)


_EC_BLK = 1280
_EC_REAL_BLOCKS = N_EDGES // _EC_BLK


def _tc_ec_body(ea_ref, we_ref, bm_ref, ec_ref):
    i = pl.program_id(0)
    v = jnp.dot(ea_ref[...], we_ref[...], preferred_element_type=jnp.float32) \
        + bm_ref[...]
    ec_ref[...] = jnp.where(i < _EC_REAL_BLOCKS, v, -1e30)


def _tc_ec(ea_p, W_e, b_msg):
    grid = (E_PAD // _EC_BLK,)
    return pl.pallas_call(
        _tc_ec_body,
        grid=grid,
        in_specs=[
            pl.BlockSpec((_EC_BLK, DE), lambda i: (i, 0)),
            pl.BlockSpec((DE, D), lambda i: (0, 0)),
            pl.BlockSpec((1, D), lambda i: (0, 0)),
        ],
        out_specs=pl.BlockSpec((_EC_BLK, D), lambda i: (i, 0)),
        out_shape=jax.ShapeDtypeStruct((E_PAD, D), jnp.float32),
    )(ea_p, W_e, b_msg)


def _merge_agg(agg_ref):
    return agg_ref[0, :N_NODES, :] + agg_ref[1, :N_NODES, :]


def _tc_node_body(h_ref, agg_ref, wnh_ref, wna_ref, bn_ref, wh_ref,
                  h_out_ref, hw_out_ref):
    a = _merge_agg(agg_ref)
    hn = jnp.maximum(
        jnp.dot(h_ref[...], wnh_ref[...], preferred_element_type=jnp.float32)
        + jnp.dot(a, wna_ref[...], preferred_element_type=jnp.float32)
        + bn_ref[...], 0.0)
    h_out_ref[...] = hn
    hw_out_ref[...] = jnp.dot(hn, wh_ref[...],
                              preferred_element_type=jnp.float32)


def _tc_node(h, agg2, Wn_h, Wn_a, b_node, Wh):
    return pl.pallas_call(
        _tc_node_body,
        out_shape=(
            jax.ShapeDtypeStruct((N_NODES, D), jnp.float32),
            jax.ShapeDtypeStruct((N_NODES, D), jnp.float32),
        ),
    )(h, agg2, Wn_h, Wn_a, b_node, Wh)


def _tc_last_body(h_ref, agg_ref, wnh_ref, wna_ref, bn_ref, wa_ref, ba_ref,
                  h_out_ref, hw_out_ref):
    a = _merge_agg(agg_ref)
    hn = jnp.maximum(
        jnp.dot(h_ref[...], wnh_ref[...], preferred_element_type=jnp.float32)
        + jnp.dot(a, wna_ref[...], preferred_element_type=jnp.float32)
        + bn_ref[...], 0.0)
    z = jnp.sum(hn * wa_ref[...], axis=1, keepdims=True) + ba_ref[...]
    aw = 1.0 / (1.0 + jnp.exp(-z))
    h_out_ref[...] = hn
    hw_out_ref[...] = hn * aw


def _tc_last(h, agg2, Wn_h, Wn_a, b_node, wa_row, ba):
    return pl.pallas_call(
        _tc_last_body,
        out_shape=(
            jax.ShapeDtypeStruct((N_NODES, D), jnp.float32),
            jax.ShapeDtypeStruct((N_NODES, D), jnp.float32),
        ),
    )(h, agg2, Wn_h, Wn_a, b_node, wa_row, ba)


def _tc_combine_body(ps_ref, pm_ref, out_ref):
    ws = jnp.sum(ps_ref[...], axis=0)
    hm = jnp.max(pm_ref[...], axis=0)
    out_ref[...] = jnp.concatenate([ws, hm], axis=1)


def _tc_combine(ps, pm):
    return pl.pallas_call(
        _tc_combine_body,
        out_shape=jax.ShapeDtypeStruct((N_GRAPHS, 2 * D), jnp.float32),
    )(ps, pm)


def kernel(x, edge_index, edge_attr, graph_ids, W_in, b_in, W_msg, b_msg,
           W_node, b_node, w_atom, b_atom):
    src = edge_index[0].astype(jnp.int32)
    dst = edge_index[1].astype(jnp.int32)
    pad = E_PAD - N_EDGES
    src_p = jnp.concatenate([src, jnp.zeros((pad,), jnp.int32)])
    dst_p = jnp.concatenate([dst, jnp.zeros((pad,), jnp.int32)])
    ea_p = jnp.concatenate(
        [edge_attr, jnp.zeros((pad, DE), edge_attr.dtype)], axis=0)
    ids_p = graph_ids.astype(jnp.int32)

    Wh = W_msg[:D]
    W_e = W_msg[D:]
    Wn_h = W_node[:D]
    Wn_a = W_node[D:]
    b_in2 = b_in.reshape(1, D)
    b_msg2 = b_msg.reshape(1, D)
    b_node2 = b_node.reshape(1, D)
    wa_row = w_atom.reshape(1, D)
    ba2 = b_atom.reshape(1, 1)

    ec = _tc_ec(ea_p, W_e, b_msg2)
    h, hw = _tc_in(x, W_in, b_in2, Wh)
    for layer in range(N_LAYERS):
        agg2 = _sc_edge(hw, ec, src_p, dst_p)
        if layer < N_LAYERS - 1:
            h, hw = _tc_node(h, agg2, Wn_h, Wn_a, b_node2, Wh)
        else:
            h, hwt = _tc_last(h, agg2, Wn_h, Wn_a, b_node2, wa_row, ba2)
    ps, pm = _sc_readout(h, hwt, ids_p)
    return _tc_combine(ps, pm)


# speedup vs baseline: 1.1443x; 1.1443x over previous
import functools

import jax
import jax.numpy as jnp
from jax import lax
from jax.experimental import pallas as pl
from jax.experimental.pallas import tpu as pltpu
from jax.experimental.pallas import tpu_sc as plsc

N_LAYERS = 4
N_NODES = 10000
N_EDGES = 320000
D = 128
DE = 16
N_GRAPHS = 200

NC = 2
NS = 16
NW = NC * NS

DH = D // NC
E_PAD = 327680
EPT = E_PAD // NS
ECHK = 256
GSUB = 128
NCHK = EPT // ECHK
ACC_ROWS = 10240
RPT = ACC_ROWS // NS

RNODES = 312
RBUF = 328
RCHK = 128
RTAIL = N_NODES - (NW - 1) * RNODES - 2 * RCHK


def _sc_edge_kernel(hw_hbm, ec_hbm, src_hbm, dst_hbm, out_hbm,
                    srcA, srcB, dstA, dstB, dsA, dsB, ecA, ecB, rowsA, rowsB,
                    acc, sIA, sIB, sGA, sGB, sEA, sEB, sSA, sSB):
    cid = lax.axis_index("c")
    sid = lax.axis_index("s")
    rowoff = cid * N_NODES

    src_bufs = (srcA, srcB)
    dst_bufs = (dstA, dstB)
    ds_bufs = (dsA, dsB)
    ec_bufs = (ecA, ecB)
    rows_bufs = (rowsA, rowsB)
    sI = (sIA, sIB)
    sG = (sGA, sGB)
    sE = (sEA, sEB)
    sS = (sSA, sSB)

    def _z(i, _):
        for j in range(DH // 16):
            rowsA[i, pl.ds(j * 16, 16)] = jnp.zeros((16,), jnp.float32)
        return 0
    lax.fori_loop(0, ECHK, _z, 0)
    pltpu.sync_copy(rowsA, acc.at[pl.ds(sid * RPT, ECHK)])
    pltpu.sync_copy(rowsA, acc.at[pl.ds(sid * RPT + ECHK, ECHK)])
    pltpu.sync_copy(rowsA.at[pl.ds(0, RPT - 2 * ECHK)],
                    acc.at[pl.ds(sid * RPT + 2 * ECHK, RPT - 2 * ECHK)])
    plsc.subcore_barrier()

    base = sid * EPT

    def issue_idx(c, b):
        e0 = base + c * ECHK
        pltpu.async_copy(src_hbm.at[pl.ds(e0, ECHK)], src_bufs[b], sI[b])
        pltpu.async_copy(dst_hbm.at[pl.ds(e0, ECHK)], dst_bufs[b], sI[b])

    def wait_idx(b):
        pltpu.make_async_copy(src_hbm.at[pl.ds(0, ECHK)], src_bufs[b],
                              sI[b]).wait()
        pltpu.make_async_copy(dst_hbm.at[pl.ds(0, ECHK)], dst_bufs[b],
                              sI[b]).wait()

    def adjust_and_issue(c, b):
        for j in range(ECHK // 16):
            sl = pl.ds(j * 16, 16)
            src_bufs[b][sl] = src_bufs[b][sl] + rowoff
        for g in range(ECHK // GSUB):
            pltpu.async_copy(
                hw_hbm.at[src_bufs[b].at[pl.ds(g * GSUB, GSUB)]],
                rows_bufs[b].at[pl.ds(g * GSUB, GSUB)], sG[b])
        e0 = base + c * ECHK
        pltpu.async_copy(ec_hbm.at[cid, pl.ds(e0, ECHK)], ec_bufs[b], sE[b])

    def wait_ge(b):
        for g in range(ECHK // GSUB):
            pltpu.make_async_copy(
                hw_hbm.at[src_bufs[b].at[pl.ds(g * GSUB, GSUB)]],
                rows_bufs[b].at[pl.ds(g * GSUB, GSUB)], sG[b]).wait()
        pltpu.make_async_copy(ec_hbm.at[cid, pl.ds(0, ECHK)], ec_bufs[b],
                              sE[b]).wait()

    def compute(b):
        def _row(i, _):
            for j in range(DH // 16):
                sl = pl.ds(j * 16, 16)
                rows_bufs[b][i, sl] = jnp.maximum(
                    rows_bufs[b][i, sl] + ec_bufs[b][i, sl], 0.0)
            return 0
        lax.fori_loop(0, ECHK, _row, 0)
        for g in range(ECHK // GSUB):
            for j in range(GSUB // 16):
                ds_bufs[b][g, pl.ds(j * 16, 16)] = \
                    dst_bufs[b][pl.ds(g * GSUB + j * 16, 16)]

    def scatter_start(b):
        for g in range(ECHK // GSUB):
            pltpu.async_copy(rows_bufs[b].at[pl.ds(g * GSUB, GSUB)],
                             acc.at[ds_bufs[b].at[g]], sS[b], add=True)

    def scatter_wait(b):
        for g in range(ECHK // GSUB):
            pltpu.make_async_copy(rows_bufs[b].at[pl.ds(g * GSUB, GSUB)],
                                  acc.at[ds_bufs[b].at[g]], sS[b]).wait()

    issue_idx(0, 0)
    wait_idx(0)
    adjust_and_issue(0, 0)
    issue_idx(1, 1)

    wait_ge(0)
    compute(0)
    wait_idx(1)
    adjust_and_issue(1, 1)
    issue_idx(2, 0)
    scatter_start(0)

    def _steady(i, _):
        for k, b in enumerate((1, 0)):
            c = 2 * i + 1 + k
            b2 = 1 - b
            wait_ge(b)
            compute(b)
            wait_idx(b2)
            scatter_wait(b2)
            adjust_and_issue(c + 1, b2)
            issue_idx(c + 2, b)
            scatter_start(b)
        return 0
    lax.fori_loop(0, (NCHK - 4) // 2, _steady, 0)

    wait_ge(1)
    compute(1)
    wait_idx(0)
    scatter_wait(0)
    adjust_and_issue(NCHK - 2, 0)
    issue_idx(NCHK - 1, 1)
    scatter_start(1)

    wait_ge(0)
    compute(0)
    wait_idx(1)
    scatter_wait(1)
    adjust_and_issue(NCHK - 1, 1)
    scatter_start(0)

    wait_ge(1)
    compute(1)
    scatter_start(1)

    scatter_wait(0)
    scatter_wait(1)
    plsc.subcore_barrier()
    pltpu.sync_copy(acc.at[pl.ds(sid * RPT, RPT)],
                    out_hbm.at[cid, pl.ds(sid * RPT, RPT)])


_sc_edge = functools.partial(
    pl.kernel,
    mesh=plsc.VectorSubcoreMesh(core_axis_name="c", subcore_axis_name="s"),
    out_type=jax.ShapeDtypeStruct((NC, ACC_ROWS, DH), jnp.float32),
    scratch_types=[
        pltpu.VMEM((ECHK,), jnp.int32),
        pltpu.VMEM((ECHK,), jnp.int32),
        pltpu.VMEM((ECHK,), jnp.int32),
        pltpu.VMEM((ECHK,), jnp.int32),
        pltpu.VMEM((ECHK // GSUB, GSUB), jnp.int32),
        pltpu.VMEM((ECHK // GSUB, GSUB), jnp.int32),
        pltpu.VMEM((ECHK, DH), jnp.float32),
        pltpu.VMEM((ECHK, DH), jnp.float32),
        pltpu.VMEM((ECHK, DH), jnp.float32),
        pltpu.VMEM((ECHK, DH), jnp.float32),
        pltpu.VMEM_SHARED((ACC_ROWS, DH), jnp.float32),
        pltpu.SemaphoreType.DMA,
        pltpu.SemaphoreType.DMA,
        pltpu.SemaphoreType.DMA,
        pltpu.SemaphoreType.DMA,
        pltpu.SemaphoreType.DMA,
        pltpu.SemaphoreType.DMA,
        pltpu.SemaphoreType.DMA,
        pltpu.SemaphoreType.DMA,
    ],
    compiler_params=pltpu.CompilerParams(needs_layout_passes=False,
                                         use_tc_tiling_on_sc=False),
)(_sc_edge_kernel)


def _sc_readout_kernel(h_hbm, hw_hbm, ids_hbm, psum_hbm, pmax_hbm,
                       hbuf, wbuf, idsbuf, pS, pM):
    cid = lax.axis_index("c")
    sid = lax.axis_index("s")
    w = sid * NC + cid
    base = w * RNODES
    cnt = jnp.where(w == NW - 1, RBUF, RNODES)

    pltpu.sync_copy(ids_hbm.at[pl.ds(base, RBUF)], idsbuf)

    def _z(i, _):
        for j in range(D // 16):
            sl = pl.ds(j * 16, 16)
            pS[i, sl] = jnp.zeros((16,), jnp.float32)
            pM[i, sl] = jnp.zeros((16,), jnp.float32)
        return 0
    lax.fori_loop(0, N_GRAPHS, _z, 0)

    for c in range(3):
        start = base + c * RCHK
        if c < 2:
            pltpu.sync_copy(h_hbm.at[pl.ds(start, RCHK)], hbuf)
            pltpu.sync_copy(hw_hbm.at[pl.ds(start, RCHK)], wbuf)
        else:
            @pl.when(w < NW - 1)
            def _full():
                pltpu.sync_copy(h_hbm.at[pl.ds(start, RCHK)], hbuf)
                pltpu.sync_copy(hw_hbm.at[pl.ds(start, RCHK)], wbuf)

            @pl.when(w == NW - 1)
            def _tail():
                pltpu.sync_copy(h_hbm.at[pl.ds(start, RTAIL)],
                                hbuf.at[pl.ds(0, RTAIL)])
                pltpu.sync_copy(hw_hbm.at[pl.ds(start, RTAIL)],
                                wbuf.at[pl.ds(0, RTAIL)])

        rem = jnp.clip(cnt - c * RCHK, 0, RCHK)

        def _node(n, _):
            g16 = plsc.load_gather(
                idsbuf, [jnp.full((16,), c * RCHK + n, jnp.int32)])
            for j in range(D // 16):
                sl = pl.ds(j * 16, 16)
                cols = lax.iota(jnp.int32, 16) + j * 16
                s_old = plsc.load_gather(pS, [g16, cols])
                plsc.store_scatter(pS, [g16, cols], s_old + wbuf[n, sl])
                m_old = plsc.load_gather(pM, [g16, cols])
                plsc.store_scatter(pM, [g16, cols],
                                   jnp.maximum(m_old, hbuf[n, sl]))
            return 0
        lax.fori_loop(0, rem, _node, 0)

    pltpu.sync_copy(pS, psum_hbm.at[w])
    pltpu.sync_copy(pM, pmax_hbm.at[w])


_sc_readout = functools.partial(
    pl.kernel,
    mesh=plsc.VectorSubcoreMesh(core_axis_name="c", subcore_axis_name="s"),
    out_type=(
        jax.ShapeDtypeStruct((NW, N_GRAPHS, D), jnp.float32),
        jax.ShapeDtypeStruct((NW, N_GRAPHS, D), jnp.float32),
    ),
    scratch_types=[
        pltpu.VMEM((RCHK, D), jnp.float32),
        pltpu.VMEM((RCHK, D), jnp.float32),
        pltpu.VMEM((RBUF,), jnp.int32),
        pltpu.VMEM((N_GRAPHS, D), jnp.float32),
        pltpu.VMEM((N_GRAPHS, D), jnp.float32),
    ],
    compiler_params=pltpu.CompilerParams(needs_layout_passes=False),
)(_sc_readout_kernel)


def _split_cols(r):
    return jnp.concatenate([r[:, :DH], r[:, DH:]], axis=0)


def _tc_in_body(x_ref, win_ref, bin_ref, wh_ref, h_ref, hw_ref):
    h = jnp.maximum(
        jnp.dot(x_ref[...], win_ref[...], preferred_element_type=jnp.float32)
        + bin_ref[...], 0.0)
    h_ref[...] = h
    hw_ref[...] = _split_cols(
        jnp.dot(h, wh_ref[...], preferred_element_type=jnp.float32))


def _tc_in(x, W_in, b_in, Wh):
    return pl.pallas_call(
        _tc_in_body,
        out_shape=(
            jax.ShapeDtypeStruct((N_NODES, D), jnp.float32),
            jax.ShapeDtypeStruct((NC * N_NODES, DH), jnp.float32),
        ),
    )(x, W_in, b_in, Wh)


_EC_BLK = 1280
_EC_REAL_BLOCKS = N_EDGES // _EC_BLK


def _tc_ec_body(ea_ref, we_ref, bm_ref, ec_ref):
    i = pl.program_id(0)
    v = jnp.dot(ea_ref[...], we_ref[...], preferred_element_type=jnp.float32) \
        + bm_ref[...]
    v = jnp.where(i < _EC_REAL_BLOCKS, v, -1e30)
    ec_ref[0] = v[:, :DH]
    ec_ref[1] = v[:, DH:]


def _tc_ec(ea_p, W_e, b_msg):
    grid = (E_PAD // _EC_BLK,)
    return pl.pallas_call(
        _tc_ec_body,
        grid=grid,
        in_specs=[
            pl.BlockSpec((_EC_BLK, DE), lambda i: (i, 0)),
            pl.BlockSpec((DE, D), lambda i: (0, 0)),
            pl.BlockSpec((1, D), lambda i: (0, 0)),
        ],
        out_specs=pl.BlockSpec((NC, _EC_BLK, DH), lambda i: (0, i, 0)),
        out_shape=jax.ShapeDtypeStruct((NC, E_PAD, DH), jnp.float32),
    )(ea_p, W_e, b_msg)


def _merge_agg(agg_ref):
    return jnp.concatenate(
        [agg_ref[0, :N_NODES, :], agg_ref[1, :N_NODES, :]], axis=1)


def _tc_node_body(h_ref, agg_ref, wnh_ref, wna_ref, bn_ref, wh_ref,
                  h_out_ref, hw_out_ref):
    a = _merge_agg(agg_ref)
    hn = jnp.maximum(
        jnp.dot(h_ref[...], wnh_ref[...], preferred_element_type=jnp.float32)
        + jnp.dot(a, wna_ref[...], preferred_element_type=jnp.float32)
        + bn_ref[...], 0.0)
    h_out_ref[...] = hn
    hw_out_ref[...] = _split_cols(
        jnp.dot(hn, wh_ref[...], preferred_element_type=jnp.float32))


def _tc_node(h, agg2, Wn_h, Wn_a, b_node, Wh):
    return pl.pallas_call(
        _tc_node_body,
        out_shape=(
            jax.ShapeDtypeStruct((N_NODES, D), jnp.float32),
            jax.ShapeDtypeStruct((NC * N_NODES, DH), jnp.float32),
        ),
    )(h, agg2, Wn_h, Wn_a, b_node, Wh)


def _tc_last_body(h_ref, agg_ref, wnh_ref, wna_ref, bn_ref, wa_ref, ba_ref,
                  h_out_ref, hw_out_ref):
    a = _merge_agg(agg_ref)
    hn = jnp.maximum(
        jnp.dot(h_ref[...], wnh_ref[...], preferred_element_type=jnp.float32)
        + jnp.dot(a, wna_ref[...], preferred_element_type=jnp.float32)
        + bn_ref[...], 0.0)
    z = jnp.sum(hn * wa_ref[...], axis=1, keepdims=True) + ba_ref[...]
    aw = 1.0 / (1.0 + jnp.exp(-z))
    h_out_ref[...] = hn
    hw_out_ref[...] = hn * aw


def _tc_last(h, agg2, Wn_h, Wn_a, b_node, wa_row, ba):
    return pl.pallas_call(
        _tc_last_body,
        out_shape=(
            jax.ShapeDtypeStruct((N_NODES, D), jnp.float32),
            jax.ShapeDtypeStruct((N_NODES, D), jnp.float32),
        ),
    )(h, agg2, Wn_h, Wn_a, b_node, wa_row, ba)


def _tc_combine_body(ps_ref, pm_ref, out_ref):
    ws = jnp.sum(ps_ref[...], axis=0)
    hm = jnp.max(pm_ref[...], axis=0)
    out_ref[...] = jnp.concatenate([ws, hm], axis=1)


def _tc_combine(ps, pm):
    return pl.pallas_call(
        _tc_combine_body,
        out_shape=jax.ShapeDtypeStruct((N_GRAPHS, 2 * D), jnp.float32),
    )(ps, pm)


def kernel(x, edge_index, edge_attr, graph_ids, W_in, b_in, W_msg, b_msg,
           W_node, b_node, w_atom, b_atom):
    src = edge_index[0].astype(jnp.int32)
    dst = edge_index[1].astype(jnp.int32)
    pad = E_PAD - N_EDGES
    src_p = jnp.concatenate([src, jnp.zeros((pad,), jnp.int32)])
    dst_p = jnp.concatenate([dst, jnp.zeros((pad,), jnp.int32)])
    ea_p = jnp.concatenate(
        [edge_attr, jnp.zeros((pad, DE), edge_attr.dtype)], axis=0)
    ids_p = graph_ids.astype(jnp.int32)

    Wh = W_msg[:D]
    W_e = W_msg[D:]
    Wn_h = W_node[:D]
    Wn_a = W_node[D:]
    b_in2 = b_in.reshape(1, D)
    b_msg2 = b_msg.reshape(1, D)
    b_node2 = b_node.reshape(1, D)
    wa_row = w_atom.reshape(1, D)
    ba2 = b_atom.reshape(1, 1)

    ec = _tc_ec(ea_p, W_e, b_msg2)
    h, hw = _tc_in(x, W_in, b_in2, Wh)
    for layer in range(N_LAYERS):
        agg2 = _sc_edge(hw, ec, src_p, dst_p)
        if layer < N_LAYERS - 1:
            h, hw = _tc_node(h, agg2, Wn_h, Wn_a, b_node2, Wh)
        else:
            h, hwt = _tc_last(h, agg2, Wn_h, Wn_a, b_node2, wa_row, ba2)
    ps, pm = _sc_readout(h, hwt, ids_p)
    return _tc_combine(ps, pm)


# speedup vs baseline: 1.5561x; 1.3598x over previous
import functools

import jax
import jax.numpy as jnp
from jax import lax
from jax.experimental import pallas as pl
from jax.experimental.pallas import tpu as pltpu
from jax.experimental.pallas import tpu_sc as plsc

N_LAYERS = 4
N_NODES = 10000
N_EDGES = 320000
D = 128
DE = 16
N_GRAPHS = 200

NC = 2
NS = 16
NW = NC * NS

DH = D // NC
E_PAD = 327680
EPT = E_PAD // NS
ECHK = 128
GSUB = 128
NCHK = EPT // ECHK
ACC_ROWS = 10240
RPT = ACC_ROWS // NS

RNODES = 312
RBUF = 328
RCHK = 128
RTAIL = N_NODES - (NW - 1) * RNODES - 2 * RCHK


def _sc_edge_kernel(hw_hbm, ec_hbm, src_hbm, dst_hbm, out_hbm,
                    srcA, srcB, dstA, dstB, dsA, dsB, ecA, ecB, rowsA, rowsB,
                    acc, hw_spm, sIA, sIB, sGA, sGB, sEA, sEB, sSA, sSB):
    cid = lax.axis_index("c")
    sid = lax.axis_index("s")

    src_bufs = (srcA, srcB)
    dst_bufs = (dstA, dstB)
    ds_bufs = (dsA, dsB)
    ec_bufs = (ecA, ecB)
    rows_bufs = (rowsA, rowsB)
    sI = (sIA, sIB)
    sG = (sGA, sGB)
    sE = (sEA, sEB)
    sS = (sSA, sSB)

    def _z(i, _):
        for j in range(DH // 16):
            rowsA[i, pl.ds(j * 16, 16)] = jnp.zeros((16,), jnp.float32)
        return 0
    lax.fori_loop(0, ECHK, _z, 0)
    pltpu.sync_copy(hw_hbm.at[cid, pl.ds(sid * RPT, RPT)],
                    hw_spm.at[pl.ds(sid * RPT, RPT)])
    for k in range(RPT // ECHK):
        pltpu.sync_copy(rowsA, acc.at[pl.ds(sid * RPT + k * ECHK, ECHK)])
    plsc.subcore_barrier()

    base = sid * EPT

    def issue_idx(c, b):
        e0 = base + c * ECHK
        pltpu.async_copy(src_hbm.at[pl.ds(e0, ECHK)], src_bufs[b], sI[b])
        pltpu.async_copy(dst_hbm.at[pl.ds(e0, ECHK)], dst_bufs[b], sI[b])

    def wait_idx(b):
        pltpu.make_async_copy(src_hbm.at[pl.ds(0, ECHK)], src_bufs[b],
                              sI[b]).wait()
        pltpu.make_async_copy(dst_hbm.at[pl.ds(0, ECHK)], dst_bufs[b],
                              sI[b]).wait()

    def adjust_and_issue(c, b):
        for g in range(ECHK // GSUB):
            pltpu.async_copy(
                hw_spm.at[src_bufs[b].at[pl.ds(g * GSUB, GSUB)]],
                rows_bufs[b].at[pl.ds(g * GSUB, GSUB)], sG[b])
        e0 = base + c * ECHK
        pltpu.async_copy(ec_hbm.at[cid, pl.ds(e0, ECHK)], ec_bufs[b], sE[b])

    def wait_ge(b):
        for g in range(ECHK // GSUB):
            pltpu.make_async_copy(
                hw_spm.at[src_bufs[b].at[pl.ds(g * GSUB, GSUB)]],
                rows_bufs[b].at[pl.ds(g * GSUB, GSUB)], sG[b]).wait()
        pltpu.make_async_copy(ec_hbm.at[cid, pl.ds(0, ECHK)], ec_bufs[b],
                              sE[b]).wait()

    def compute(b):
        def _row(i, _):
            for j in range(DH // 16):
                sl = pl.ds(j * 16, 16)
                rows_bufs[b][i, sl] = jnp.maximum(
                    rows_bufs[b][i, sl] + ec_bufs[b][i, sl], 0.0)
            return 0
        lax.fori_loop(0, ECHK, _row, 0)
        for g in range(ECHK // GSUB):
            for j in range(GSUB // 16):
                ds_bufs[b][g, pl.ds(j * 16, 16)] = \
                    dst_bufs[b][pl.ds(g * GSUB + j * 16, 16)]

    def scatter_start(b):
        for g in range(ECHK // GSUB):
            pltpu.async_copy(rows_bufs[b].at[pl.ds(g * GSUB, GSUB)],
                             acc.at[ds_bufs[b].at[g]], sS[b], add=True)

    def scatter_wait(b):
        for g in range(ECHK // GSUB):
            pltpu.make_async_copy(rows_bufs[b].at[pl.ds(g * GSUB, GSUB)],
                                  acc.at[ds_bufs[b].at[g]], sS[b]).wait()

    issue_idx(0, 0)
    wait_idx(0)
    adjust_and_issue(0, 0)
    issue_idx(1, 1)

    wait_ge(0)
    compute(0)
    wait_idx(1)
    adjust_and_issue(1, 1)
    issue_idx(2, 0)
    scatter_start(0)

    def _steady(i, _):
        for k, b in enumerate((1, 0)):
            c = 2 * i + 1 + k
            b2 = 1 - b
            wait_ge(b)
            compute(b)
            wait_idx(b2)
            scatter_wait(b2)
            adjust_and_issue(c + 1, b2)
            issue_idx(c + 2, b)
            scatter_start(b)
        return 0
    lax.fori_loop(0, (NCHK - 4) // 2, _steady, 0)

    wait_ge(1)
    compute(1)
    wait_idx(0)
    scatter_wait(0)
    adjust_and_issue(NCHK - 2, 0)
    issue_idx(NCHK - 1, 1)
    scatter_start(1)

    wait_ge(0)
    compute(0)
    wait_idx(1)
    scatter_wait(1)
    adjust_and_issue(NCHK - 1, 1)
    scatter_start(0)

    wait_ge(1)
    compute(1)
    scatter_start(1)

    scatter_wait(0)
    scatter_wait(1)
    plsc.subcore_barrier()
    pltpu.sync_copy(acc.at[pl.ds(sid * RPT, RPT)],
                    out_hbm.at[cid, pl.ds(sid * RPT, RPT)])


_sc_edge = functools.partial(
    pl.kernel,
    mesh=plsc.VectorSubcoreMesh(core_axis_name="c", subcore_axis_name="s"),
    out_type=jax.ShapeDtypeStruct((NC, ACC_ROWS, DH), jnp.float32),
    scratch_types=[
        pltpu.VMEM((ECHK,), jnp.int32),
        pltpu.VMEM((ECHK,), jnp.int32),
        pltpu.VMEM((ECHK,), jnp.int32),
        pltpu.VMEM((ECHK,), jnp.int32),
        pltpu.VMEM((ECHK // GSUB, GSUB), jnp.int32),
        pltpu.VMEM((ECHK // GSUB, GSUB), jnp.int32),
        pltpu.VMEM((ECHK, DH), jnp.float32),
        pltpu.VMEM((ECHK, DH), jnp.float32),
        pltpu.VMEM((ECHK, DH), jnp.float32),
        pltpu.VMEM((ECHK, DH), jnp.float32),
        pltpu.VMEM_SHARED((ACC_ROWS, DH), jnp.float32),
        pltpu.VMEM_SHARED((ACC_ROWS, DH), jnp.float32),
        pltpu.SemaphoreType.DMA,
        pltpu.SemaphoreType.DMA,
        pltpu.SemaphoreType.DMA,
        pltpu.SemaphoreType.DMA,
        pltpu.SemaphoreType.DMA,
        pltpu.SemaphoreType.DMA,
        pltpu.SemaphoreType.DMA,
        pltpu.SemaphoreType.DMA,
    ],
    compiler_params=pltpu.CompilerParams(needs_layout_passes=False,
                                         use_tc_tiling_on_sc=False),
)(_sc_edge_kernel)


def _sc_readout_kernel(h_hbm, hw_hbm, ids_hbm, psum_hbm, pmax_hbm,
                       hbuf, wbuf, idsbuf, pS, pM):
    cid = lax.axis_index("c")
    sid = lax.axis_index("s")
    w = sid * NC + cid
    base = w * RNODES
    cnt = jnp.where(w == NW - 1, RBUF, RNODES)

    pltpu.sync_copy(ids_hbm.at[pl.ds(base, RBUF)], idsbuf)

    def _z(i, _):
        for j in range(D // 16):
            sl = pl.ds(j * 16, 16)
            pS[i, sl] = jnp.zeros((16,), jnp.float32)
            pM[i, sl] = jnp.zeros((16,), jnp.float32)
        return 0
    lax.fori_loop(0, N_GRAPHS, _z, 0)

    for c in range(3):
        start = base + c * RCHK
        if c < 2:
            pltpu.sync_copy(h_hbm.at[pl.ds(start, RCHK)], hbuf)
            pltpu.sync_copy(hw_hbm.at[pl.ds(start, RCHK)], wbuf)
        else:
            @pl.when(w < NW - 1)
            def _full():
                pltpu.sync_copy(h_hbm.at[pl.ds(start, RCHK)], hbuf)
                pltpu.sync_copy(hw_hbm.at[pl.ds(start, RCHK)], wbuf)

            @pl.when(w == NW - 1)
            def _tail():
                pltpu.sync_copy(h_hbm.at[pl.ds(start, RTAIL)],
                                hbuf.at[pl.ds(0, RTAIL)])
                pltpu.sync_copy(hw_hbm.at[pl.ds(start, RTAIL)],
                                wbuf.at[pl.ds(0, RTAIL)])

        rem = jnp.clip(cnt - c * RCHK, 0, RCHK)

        def _node(n, _):
            g16 = plsc.load_gather(
                idsbuf, [jnp.full((16,), c * RCHK + n, jnp.int32)])
            for j in range(D // 16):
                sl = pl.ds(j * 16, 16)
                cols = lax.iota(jnp.int32, 16) + j * 16
                s_old = plsc.load_gather(pS, [g16, cols])
                plsc.store_scatter(pS, [g16, cols], s_old + wbuf[n, sl])
                m_old = plsc.load_gather(pM, [g16, cols])
                plsc.store_scatter(pM, [g16, cols],
                                   jnp.maximum(m_old, hbuf[n, sl]))
            return 0
        lax.fori_loop(0, rem, _node, 0)

    pltpu.sync_copy(pS, psum_hbm.at[w])
    pltpu.sync_copy(pM, pmax_hbm.at[w])


_sc_readout = functools.partial(
    pl.kernel,
    mesh=plsc.VectorSubcoreMesh(core_axis_name="c", subcore_axis_name="s"),
    out_type=(
        jax.ShapeDtypeStruct((NW, N_GRAPHS, D), jnp.float32),
        jax.ShapeDtypeStruct((NW, N_GRAPHS, D), jnp.float32),
    ),
    scratch_types=[
        pltpu.VMEM((RCHK, D), jnp.float32),
        pltpu.VMEM((RCHK, D), jnp.float32),
        pltpu.VMEM((RBUF,), jnp.int32),
        pltpu.VMEM((N_GRAPHS, D), jnp.float32),
        pltpu.VMEM((N_GRAPHS, D), jnp.float32),
    ],
    compiler_params=pltpu.CompilerParams(needs_layout_passes=False),
)(_sc_readout_kernel)


def _store_split(hw_ref, r):
    hw_ref[0, :N_NODES, :] = r[:, :DH]
    hw_ref[1, :N_NODES, :] = r[:, DH:]


def _tc_in_body(x_ref, win_ref, bin_ref, wh_ref, h_ref, hw_ref):
    h = jnp.maximum(
        jnp.dot(x_ref[...], win_ref[...], preferred_element_type=jnp.float32)
        + bin_ref[...], 0.0)
    h_ref[...] = h
    _store_split(hw_ref,
                 jnp.dot(h, wh_ref[...], preferred_element_type=jnp.float32))


def _tc_in(x, W_in, b_in, Wh):
    return pl.pallas_call(
        _tc_in_body,
        out_shape=(
            jax.ShapeDtypeStruct((N_NODES, D), jnp.float32),
            jax.ShapeDtypeStruct((NC, ACC_ROWS, DH), jnp.float32),
        ),
    )(x, W_in, b_in, Wh)


_EC_BLK = 1280
_EC_REAL_BLOCKS = N_EDGES // _EC_BLK


def _tc_ec_body(ea_ref, we_ref, bm_ref, ec_ref):
    i = pl.program_id(0)
    v = jnp.dot(ea_ref[...], we_ref[...], preferred_element_type=jnp.float32) \
        + bm_ref[...]
    v = jnp.where(i < _EC_REAL_BLOCKS, v, -1e30)
    ec_ref[0] = v[:, :DH]
    ec_ref[1] = v[:, DH:]


def _tc_ec(ea_p, W_e, b_msg):
    grid = (E_PAD // _EC_BLK,)
    return pl.pallas_call(
        _tc_ec_body,
        grid=grid,
        in_specs=[
            pl.BlockSpec((_EC_BLK, DE), lambda i: (i, 0)),
            pl.BlockSpec((DE, D), lambda i: (0, 0)),
            pl.BlockSpec((1, D), lambda i: (0, 0)),
        ],
        out_specs=pl.BlockSpec((NC, _EC_BLK, DH), lambda i: (0, i, 0)),
        out_shape=jax.ShapeDtypeStruct((NC, E_PAD, DH), jnp.float32),
    )(ea_p, W_e, b_msg)


def _merge_agg(agg_ref):
    return jnp.concatenate(
        [agg_ref[0, :N_NODES, :], agg_ref[1, :N_NODES, :]], axis=1)


def _tc_node_body(h_ref, agg_ref, wnh_ref, wna_ref, bn_ref, wh_ref,
                  h_out_ref, hw_out_ref):
    a = _merge_agg(agg_ref)
    hn = jnp.maximum(
        jnp.dot(h_ref[...], wnh_ref[...], preferred_element_type=jnp.float32)
        + jnp.dot(a, wna_ref[...], preferred_element_type=jnp.float32)
        + bn_ref[...], 0.0)
    h_out_ref[...] = hn
    _store_split(hw_out_ref,
                 jnp.dot(hn, wh_ref[...], preferred_element_type=jnp.float32))


def _tc_node(h, agg2, Wn_h, Wn_a, b_node, Wh):
    return pl.pallas_call(
        _tc_node_body,
        out_shape=(
            jax.ShapeDtypeStruct((N_NODES, D), jnp.float32),
            jax.ShapeDtypeStruct((NC, ACC_ROWS, DH), jnp.float32),
        ),
    )(h, agg2, Wn_h, Wn_a, b_node, Wh)


def _tc_last_body(h_ref, agg_ref, wnh_ref, wna_ref, bn_ref, wa_ref, ba_ref,
                  h_out_ref, hw_out_ref):
    a = _merge_agg(agg_ref)
    hn = jnp.maximum(
        jnp.dot(h_ref[...], wnh_ref[...], preferred_element_type=jnp.float32)
        + jnp.dot(a, wna_ref[...], preferred_element_type=jnp.float32)
        + bn_ref[...], 0.0)
    z = jnp.sum(hn * wa_ref[...], axis=1, keepdims=True) + ba_ref[...]
    aw = 1.0 / (1.0 + jnp.exp(-z))
    h_out_ref[...] = hn
    hw_out_ref[...] = hn * aw


def _tc_last(h, agg2, Wn_h, Wn_a, b_node, wa_row, ba):
    return pl.pallas_call(
        _tc_last_body,
        out_shape=(
            jax.ShapeDtypeStruct((N_NODES, D), jnp.float32),
            jax.ShapeDtypeStruct((N_NODES, D), jnp.float32),
        ),
    )(h, agg2, Wn_h, Wn_a, b_node, wa_row, ba)


def _tc_combine_body(ps_ref, pm_ref, out_ref):
    ws = jnp.sum(ps_ref[...], axis=0)
    hm = jnp.max(pm_ref[...], axis=0)
    out_ref[...] = jnp.concatenate([ws, hm], axis=1)


def _tc_combine(ps, pm):
    return pl.pallas_call(
        _tc_combine_body,
        out_shape=jax.ShapeDtypeStruct((N_GRAPHS, 2 * D), jnp.float32),
    )(ps, pm)


def kernel(x, edge_index, edge_attr, graph_ids, W_in, b_in, W_msg, b_msg,
           W_node, b_node, w_atom, b_atom):
    src = edge_index[0].astype(jnp.int32)
    dst = edge_index[1].astype(jnp.int32)
    pad = E_PAD - N_EDGES
    src_p = jnp.concatenate([src, jnp.zeros((pad,), jnp.int32)])
    dst_p = jnp.concatenate([dst, jnp.zeros((pad,), jnp.int32)])
    ea_p = jnp.concatenate(
        [edge_attr, jnp.zeros((pad, DE), edge_attr.dtype)], axis=0)
    ids_p = graph_ids.astype(jnp.int32)

    Wh = W_msg[:D]
    W_e = W_msg[D:]
    Wn_h = W_node[:D]
    Wn_a = W_node[D:]
    b_in2 = b_in.reshape(1, D)
    b_msg2 = b_msg.reshape(1, D)
    b_node2 = b_node.reshape(1, D)
    wa_row = w_atom.reshape(1, D)
    ba2 = b_atom.reshape(1, 1)

    ec = _tc_ec(ea_p, W_e, b_msg2)
    h, hw = _tc_in(x, W_in, b_in2, Wh)
    for layer in range(N_LAYERS):
        agg2 = _sc_edge(hw, ec, src_p, dst_p)
        if layer < N_LAYERS - 1:
            h, hw = _tc_node(h, agg2, Wn_h, Wn_a, b_node2, Wh)
        else:
            h, hwt = _tc_last(h, agg2, Wn_h, Wn_a, b_node2, wa_row, ba2)
    ps, pm = _sc_readout(h, hwt, ids_p)
    return _tc_combine(ps, pm)
